# Initial kernel scaffold; baseline (speedup 1.0000x reference)
import jax
import jax.numpy as jnp
from jax.experimental import pallas as pl


def kernel(x, wt, bias):
    raise NotImplementedError("write your pallas kernel here")


# speedup vs baseline: 7.7173x; 7.7173x over previous
import jax
import jax.numpy as jnp
from jax.experimental import pallas as pl
from jax.experimental.pallas import tpu as pltpu


def _linear_kernel(x_ref, w_ref, b_ref, o_ref):
    o_ref[...] = (
        jnp.dot(x_ref[...], w_ref[...], preferred_element_type=jnp.float32)
        + b_ref[...]
    )


def _largest_divisor_tile(dim, base, align):
    t = min(base, dim)
    t = max((t // align) * align, align)
    while dim % t:
        t -= align
    return t


def kernel(x, wt, bias):
    k_pad, n_pad = wt.shape
    orig_lead = x.shape[:-1]
    feat = x.shape[-1]
    assert feat == k_pad, "activations must match the prepadded weight K"

    x2d = x.reshape(-1, feat)
    m = x2d.shape[0]

    xb = x2d.astype(jnp.bfloat16)
    wb = wt.astype(jnp.bfloat16)
    b32 = bias.astype(jnp.float32)

    tn = n_pad // 2 if (n_pad // 2) % 128 == 0 else n_pad
    tm = _largest_divisor_tile(m, 512, 8)
    grid = (n_pad // tn, m // tm)

    cost = pl.CostEstimate(
        flops=2 * m * k_pad * n_pad,
        transcendentals=0,
        bytes_accessed=(m * k_pad + k_pad * n_pad) * 2 + (n_pad + m * n_pad) * 4,
    )

    y2d = pl.pallas_call(
        _linear_kernel,
        out_shape=jax.ShapeDtypeStruct((m, n_pad), jnp.float32),
        grid_spec=pltpu.PrefetchScalarGridSpec(
            num_scalar_prefetch=0,
            grid=grid,
            in_specs=[
                pl.BlockSpec((tm, k_pad), lambda j, i: (i, 0)),
                pl.BlockSpec((k_pad, tn), lambda j, i: (0, j)),
                pl.BlockSpec((1, tn), lambda j, i: (0, j)),
            ],
            out_specs=pl.BlockSpec((tm, tn), lambda j, i: (i, j)),
        ),
        compiler_params=pltpu.CompilerParams(
            dimension_semantics=("parallel", "arbitrary"),
            vmem_limit_bytes=56 * 1024 * 1024,
        ),
        cost_estimate=cost,
    )(xb, wb, b32)

    return y2d.reshape(*orig_lead, n_pad)


# speedup vs baseline: 9.0124x; 1.1678x over previous
import jax
import jax.numpy as jnp
from jax.experimental import pallas as pl
from jax.experimental.pallas import tpu as pltpu

_STRIP = 1024


def _linear_kernel(x_ref, w_ref, b_ref, o_hbm, acc_ref, sem):
    j = pl.program_id(0)
    k = pl.program_id(1)
    nk = pl.num_programs(1)
    tm, n = acc_ref.shape

    @pl.when(k == 0)
    def _():
        acc_ref[...] = jnp.broadcast_to(b_ref[...], acc_ref.shape)

    xb = x_ref[...].astype(jnp.bfloat16)
    for s in range(0, n, _STRIP):
        sl = pl.ds(s, min(_STRIP, n - s))
        acc_ref[:, sl] += jnp.dot(
            xb,
            w_ref[:, sl].astype(jnp.bfloat16),
            preferred_element_type=jnp.float32,
        )

    @pl.when(k == nk - 1)
    def _():
        cp = pltpu.make_async_copy(
            acc_ref, o_hbm.at[pl.ds(j * tm, tm), :], sem
        )
        cp.start()
        cp.wait()


def kernel(x, wt, bias):
    k_pad, n_pad = wt.shape
    orig_lead = x.shape[:-1]
    feat = x.shape[-1]
    assert feat == k_pad, "activations must match the prepadded weight K"

    x2d = x.reshape(-1, feat)
    m = x2d.shape[0]
    b32 = bias.astype(jnp.float32)

    tm = m // 2 if (m // 2) % 8 == 0 else m
    tk = 256
    while k_pad % tk:
        tk //= 2
    grid = (m // tm, k_pad // tk)

    cost = pl.CostEstimate(
        flops=2 * m * k_pad * n_pad,
        transcendentals=0,
        bytes_accessed=(m * k_pad + k_pad * n_pad + n_pad + m * n_pad) * 4,
    )

    y2d = pl.pallas_call(
        _linear_kernel,
        out_shape=jax.ShapeDtypeStruct((m, n_pad), jnp.float32),
        grid_spec=pltpu.PrefetchScalarGridSpec(
            num_scalar_prefetch=0,
            grid=grid,
            in_specs=[
                pl.BlockSpec((tm, tk), lambda j, k: (j, k)),
                pl.BlockSpec((tk, n_pad), lambda j, k: (k, 0)),
                pl.BlockSpec((1, n_pad), lambda j, k: (0, 0)),
            ],
            out_specs=pl.BlockSpec(memory_space=pl.ANY),
            scratch_shapes=[
                pltpu.VMEM((tm, n_pad), jnp.float32),
                pltpu.SemaphoreType.DMA,
            ],
        ),
        compiler_params=pltpu.CompilerParams(
            dimension_semantics=("parallel", "arbitrary"),
            vmem_limit_bytes=56 * 1024 * 1024,
        ),
        cost_estimate=cost,
    )(x2d, wt, b32)

    return y2d.reshape(*orig_lead, n_pad)
